# baseline (speedup 1.0000x reference)
import functools

import jax
import jax.numpy as jnp
from jax import lax
from jax.experimental import pallas as pl
from jax.experimental.pallas import tpu as pltpu
from jax.experimental.pallas import tpu_sc as plsc

_N_EMB = 8192
_DIM = 256
_ROWS = 8192
_BLK = 256
_NBLK = _ROWS // _BLK


def _score_argmin_body(x2_ref, w_ref, s1_ref, s2_ref, idx_ref):
    mm2 = lax.dot_general(
        x2_ref[...], w_ref[...], (((1,), (1,)), ((), ())),
        precision="default", preferred_element_type=jnp.float32)
    scores = (s1_ref[...] + s2_ref[...]) - mm2
    dmin = jnp.min(scores, axis=1, keepdims=True)
    col = lax.broadcasted_iota(jnp.int32, scores.shape, 1)
    idx = jnp.min(jnp.where(scores == dmin, col, jnp.int32(_N_EMB)), axis=1)
    idx_ref[0, 0, :] = idx


def _tc_argmin(x2, W, s1, s2):
    return pl.pallas_call(
        _score_argmin_body,
        grid=(_NBLK,),
        in_specs=[
            pl.BlockSpec((_BLK, _DIM), lambda i: (i, 0)),
            pl.BlockSpec((_N_EMB, _DIM), lambda i: (0, 0)),
            pl.BlockSpec((_BLK, 1), lambda i: (i, 0)),
            pl.BlockSpec((1, _N_EMB), lambda i: (0, 0)),
        ],
        out_specs=pl.BlockSpec((1, 1, _BLK), lambda i: (i, 0, 0)),
        out_shape=jax.ShapeDtypeStruct((_NBLK, 1, _BLK), jnp.int32),
    )(x2, W, s1, s2)


_IDXCHUNK = 128


def _sc_gather(W, idx):
    info = plsc.get_sparse_core_info()
    num_workers = info.num_cores * info.num_subcores
    bpw = _ROWS // num_workers
    nchunk = bpw // _IDXCHUNK
    idx2 = idx.reshape(_ROWS // _IDXCHUNK, _IDXCHUNK)
    mesh = plsc.VectorSubcoreMesh(core_axis_name="c", subcore_axis_name="s")

    @functools.partial(
        pl.kernel, mesh=mesh,
        out_type=jax.ShapeDtypeStruct((_ROWS, _DIM), jnp.float32),
        scratch_types=[
            pltpu.VMEM((nchunk, _IDXCHUNK), jnp.int32),
            pltpu.VMEM((nchunk, _IDXCHUNK, _DIM), jnp.float32),
            pltpu.SemaphoreType.DMA,
        ],
    )
    def gather_kernel(table_hbm, idx_hbm, out_hbm, idx_v, rows_v, sem):
        wid = lax.axis_index("s") * info.num_cores + lax.axis_index("c")
        pltpu.sync_copy(idx_hbm.at[pl.ds(wid * nchunk, nchunk)], idx_v)
        copies = [pltpu.async_copy(table_hbm.at[idx_v.at[j]], rows_v.at[j], sem)
                  for j in range(nchunk)]
        for c in copies:
            c.wait()
        for j in range(nchunk):
            pltpu.sync_copy(
                rows_v.at[j],
                out_hbm.at[pl.ds(wid * bpw + j * _IDXCHUNK, _IDXCHUNK)])

    return gather_kernel(W, idx2)


def _loss_body(q_ref, x_ref, out_ref):
    diff = q_ref[...] - x_ref[...]
    out_ref[0, 0, :] = jnp.sum(diff * diff, axis=0)


def _tc_loss_partials(q, x_flat):
    return pl.pallas_call(
        _loss_body,
        grid=(_NBLK,),
        in_specs=[
            pl.BlockSpec((_BLK, _DIM), lambda i: (i, 0)),
            pl.BlockSpec((_BLK, _DIM), lambda i: (i, 0)),
        ],
        out_specs=pl.BlockSpec((1, 1, _DIM), lambda i: (i, 0, 0)),
        out_shape=jax.ShapeDtypeStruct((_NBLK, 1, _DIM), jnp.float32),
    )(q, x_flat)


def kernel(x, W):
    x_flat = x.reshape(-1, _DIM)
    s1 = jnp.sum(x_flat ** 2, axis=1, keepdims=True)
    s2 = jnp.sum(W ** 2, axis=1)[None, :]
    idx3 = _tc_argmin(x_flat * 2.0, W, s1, s2)
    idx = lax.optimization_barrier(idx3.reshape(_ROWS))
    q = _sc_gather(W, idx)
    parts = _tc_loss_partials(q, x_flat)
    mean = jnp.sum(parts) / jnp.float32(x.size)
    loss = mean + 0.25 * mean
    quantized_st = x + lax.stop_gradient(q.reshape(x.shape) - x)
    return (quantized_st, loss)
